# baseline (speedup 1.0000x reference)
import jax
import jax.numpy as jnp
from jax import lax
from jax.experimental import pallas as pl
from jax.experimental.pallas import tpu as pltpu
from jax.experimental.pallas import tpu_sc as plsc

DIM = 768
HEADS = 12
NUM_KV = 784
NUM_KEYS = 28
TOPK = 8
DIM_KEY = 48
S = 2048
S_BLK_R = 512
S_BLK_G = 512
S_BLK_A = 256
PAD = 896

_NEG = -1e30
_HIGHEST = lax.Precision.HIGHEST


def _top8(c):
    j = lax.broadcasted_iota(jnp.int32, c.shape, 1)
    vals, idxs = [], []
    for _ in range(TOPK):
        m = jnp.max(c, axis=1, keepdims=True)
        sel = jnp.min(jnp.where(c == m, j, jnp.int32(1 << 30)), axis=1, keepdims=True)
        vals.append(m)
        idxs.append(sel)
        c = jnp.where(j == sel, _NEG, c)
    return jnp.concatenate(vals, axis=1), jnp.concatenate(idxs, axis=1)


def _router_body(x_ref, pkw0_ref, pkw1_ref, k0_ref, k1_ref, e0_ref, e1_ref,
                 scores_ref, idx_ref, wwide_ref):
    x = x_ref[...].astype(jnp.bfloat16)
    qpk0 = jnp.dot(x, pkw0_ref[0], preferred_element_type=jnp.float32)
    qpk1 = jnp.dot(x, pkw1_ref[0], preferred_element_type=jnp.float32)
    sim0 = jnp.dot(qpk0.astype(jnp.bfloat16), k0_ref[0],
                   preferred_element_type=jnp.float32)
    sim1 = jnp.dot(qpk1.astype(jnp.bfloat16), k1_ref[0],
                   preferred_element_type=jnp.float32)

    c = (jnp.dot(sim0, e0_ref[...], precision=_HIGHEST, preferred_element_type=jnp.float32)
         + jnp.dot(sim1, e1_ref[...], precision=_HIGHEST, preferred_element_type=jnp.float32))
    j2 = lax.broadcasted_iota(jnp.int32, (S_BLK_R, PAD), 1)
    c = jnp.where(j2 < NUM_KV, c, _NEG)

    jf = j2.astype(jnp.float32)
    vals, idxs = [], []
    for _ in range(TOPK):
        m = jnp.max(c, axis=1, keepdims=True)
        eq = c == m
        idx_f = jnp.min(jnp.where(eq, jf, 1e9), axis=1, keepdims=True)
        c = jnp.where(eq, _NEG, c)
        vals.append(m)
        idxs.append(idx_f)
    scores = jnp.concatenate(vals, axis=1)
    idx = jnp.concatenate(idxs, axis=1).astype(jnp.int32)

    m8 = jnp.max(scores, axis=1, keepdims=True)
    e = jnp.exp(scores - m8)
    p = e / jnp.sum(e, axis=1, keepdims=True)
    scores_ref[0] = p
    idx_ref[0] = idx + pl.program_id(0) * NUM_KV
    wwide_ref[0] = jnp.concatenate(
        [jnp.broadcast_to(p[:, t:t + 1], (S_BLK_R, SC_LANES)) for t in range(TOPK)],
        axis=1)


SC_NW = 32
SC_BAGS = HEADS * S
SC_BPW = SC_BAGS // SC_NW
SC_LANES = 16
SC_CH = 8
SC_NCH = SC_BPW // SC_CH


def _sc_gather_body(idx_hbm, w_hbm, kt_hbm, kout_hbm,
                    idx_v, w_v, rowsk_v, outk_v,
                    semk0, semk1, semo, semw0, semw1):
    wid = lax.axis_index("s") * 2 + lax.axis_index("c")
    base = wid * SC_BPW
    pltpu.sync_copy(idx_hbm.at[pl.ds(base * TOPK, SC_BPW * TOPK)], idx_v)
    semks = (semk0, semk1)
    semws = (semw0, semw1)

    def fire(ci, buf):
        sl = idx_v.at[pl.ds(ci * SC_CH * TOPK, SC_CH * TOPK)]
        ck = pltpu.async_copy(kt_hbm.at[sl], rowsk_v.at[buf], semks[buf])
        cw = pltpu.async_copy(
            w_hbm.at[pl.ds((base + ci * SC_CH) * TOPK, SC_CH * TOPK), :],
            w_v.at[buf], semws[buf])
        return ck, cw

    def consume(ci, buf):
        ws = [w_v[buf, l, :] for l in range(SC_CH * TOPK)]

        @pl.when(ci > 0)
        def _():
            pltpu.make_async_copy(outk_v, kout_hbm.at[pl.ds(base, SC_CH)], semo).wait()

        def dchunk(c, _):
            for u in range(2):
                cc = c * 2 + u
                for b in range(SC_CH):
                    acck = ws[b * TOPK] * rowsk_v[buf, b * TOPK, pl.ds(cc * SC_LANES, SC_LANES)]
                    for t in range(1, TOPK):
                        w = ws[b * TOPK + t]
                        acck = acck + w * rowsk_v[buf, b * TOPK + t, pl.ds(cc * SC_LANES, SC_LANES)]
                    outk_v[b, pl.ds(cc * SC_LANES, SC_LANES)] = acck
            return 0

        lax.fori_loop(0, DIM // SC_LANES // 2, dchunk, 0)
        return pltpu.async_copy(outk_v, kout_hbm.at[pl.ds(base + ci * SC_CH, SC_CH)], semo)

    fire(0, 0)

    def pair(i, _):
        ci0 = i * 2
        fire(ci0 + 1, 1)
        pltpu.make_async_copy(kt_hbm.at[idx_v.at[pl.ds(0, SC_CH * TOPK)]],
                              rowsk_v.at[0], semk0).wait()
        pltpu.make_async_copy(w_hbm.at[pl.ds(0, SC_CH * TOPK), :],
                              w_v.at[0], semw0).wait()

        consume(ci0, 0)

        @pl.when(i < SC_NCH // 2 - 1)
        def _():
            fire(ci0 + 2, 0)

        pltpu.make_async_copy(kt_hbm.at[idx_v.at[pl.ds(0, SC_CH * TOPK)]],
                              rowsk_v.at[1], semk1).wait()
        pltpu.make_async_copy(w_hbm.at[pl.ds(0, SC_CH * TOPK), :],
                              w_v.at[1], semw1).wait()

        consume(ci0 + 1, 1)
        return 0

    lax.fori_loop(0, SC_NCH // 2, pair, 0)
    pltpu.make_async_copy(outk_v, kout_hbm.at[pl.ds(base, SC_CH)], semo).wait()


def _vbuild_body(idx_ref, scores_ref, vt_ref, v_ref):
    kv_iota = lax.broadcasted_iota(jnp.int32, (S_BLK_G, NUM_KV), 1)
    a = jnp.zeros((S_BLK_G, NUM_KV), jnp.float32)
    idxl = idx_ref[0] - pl.program_id(0) * NUM_KV
    sc = scores_ref[0]
    for t in range(TOPK):
        a = a + jnp.where(kv_iota == idxl[:, t:t + 1], sc[:, t:t + 1], 0.0)
    ab = a.astype(jnp.bfloat16)
    vt = vt_ref[...].astype(jnp.bfloat16)
    v_ref[0] = jnp.dot(ab, vt, preferred_element_type=jnp.float32).astype(jnp.bfloat16)


def _attn_body(x_ref, wq_ref, k_ref, v_ref, wout_ref, out_ref, sim_ref):
    h = pl.program_id(0)
    qb = pl.program_id(1)
    nkb = qb + 1
    q = jnp.dot(x_ref[...].astype(jnp.bfloat16), wq_ref[...].astype(jnp.bfloat16),
                preferred_element_type=jnp.float32)
    q = (q * (DIM ** -0.5)).astype(jnp.bfloat16)

    def qk_step(kb, _):
        kblk = k_ref[0, pl.ds(kb * S_BLK_A, S_BLK_A), :].astype(jnp.bfloat16)
        sim_ref[:, pl.ds(kb * S_BLK_A, S_BLK_A)] = lax.dot_general(
            q, kblk, (((1,), (1,)), ((), ())), preferred_element_type=jnp.float32)
        return 0

    lax.fori_loop(0, nkb, qk_step, 0)

    col = lax.broadcasted_iota(jnp.int32, (S_BLK_A, S), 1)
    row = lax.broadcasted_iota(jnp.int32, (S_BLK_A, S), 0) + qb * S_BLK_A
    s = jnp.where(col > row, _NEG, sim_ref[...])
    m = jnp.max(s, axis=1, keepdims=True)
    p = jnp.exp(s - m)
    p = p / jnp.sum(p, axis=1, keepdims=True)
    sim_ref[...] = p

    def av_step(kb, o):
        pblk = sim_ref[:, pl.ds(kb * S_BLK_A, S_BLK_A)].astype(jnp.bfloat16)
        vblk = v_ref[0, pl.ds(kb * S_BLK_A, S_BLK_A), :].astype(jnp.bfloat16)
        return o + jnp.dot(pblk, vblk, preferred_element_type=jnp.float32)

    o = lax.fori_loop(0, nkb, av_step, jnp.zeros((S_BLK_A, DIM), jnp.float32))
    proj = jnp.dot(o.astype(jnp.bfloat16), wout_ref[...].astype(jnp.bfloat16),
                   preferred_element_type=jnp.float32)

    @pl.when(h == 0)
    def _():
        out_ref[pl.ds(qb * S_BLK_A, S_BLK_A), :] = proj

    @pl.when(h > 0)
    def _():
        out_ref[pl.ds(qb * S_BLK_A, S_BLK_A), :] += proj


@jax.jit
def kernel(inputs, Wq, keys_emb, values_emb, pk_Wq, pk_keys, Wout):
    x = inputs[0]
    pk_keys_t = jnp.transpose(pk_keys, (0, 2, 3, 1)).astype(jnp.bfloat16)
    pkw = jnp.transpose(pk_Wq.reshape(DIM, 2, HEADS, DIM_KEY),
                        (1, 2, 0, 3)).astype(jnp.bfloat16)

    r = lax.broadcasted_iota(jnp.int32, (NUM_KEYS, PAD), 0)
    j = lax.broadcasted_iota(jnp.int32, (NUM_KEYS, PAD), 1)
    valid = j < NUM_KV
    e0 = ((j % NUM_KEYS == r) & valid).astype(jnp.float32)
    e1 = ((j // NUM_KEYS == r) & valid).astype(jnp.float32)

    scores, idx, wwide = pl.pallas_call(
        _router_body,
        grid=(HEADS, S // S_BLK_R),
        in_specs=[
            pl.BlockSpec((S_BLK_R, DIM), lambda h, sb: (sb, 0)),
            pl.BlockSpec((1, DIM, DIM_KEY), lambda h, sb: (h, 0, 0)),
            pl.BlockSpec((1, DIM, DIM_KEY), lambda h, sb: (h, 0, 0)),
            pl.BlockSpec((1, DIM_KEY, NUM_KEYS), lambda h, sb: (h, 0, 0)),
            pl.BlockSpec((1, DIM_KEY, NUM_KEYS), lambda h, sb: (h, 0, 0)),
            pl.BlockSpec((NUM_KEYS, PAD), lambda h, sb: (0, 0)),
            pl.BlockSpec((NUM_KEYS, PAD), lambda h, sb: (0, 0)),
        ],
        out_specs=[
            pl.BlockSpec((1, S_BLK_R, TOPK), lambda h, sb: (h, sb, 0)),
            pl.BlockSpec((1, S_BLK_R, TOPK), lambda h, sb: (h, sb, 0)),
            pl.BlockSpec((1, S_BLK_R, TOPK * SC_LANES), lambda h, sb: (h, sb, 0)),
        ],
        out_shape=[
            jax.ShapeDtypeStruct((HEADS, S, TOPK), jnp.float32),
            jax.ShapeDtypeStruct((HEADS, S, TOPK), jnp.int32),
            jax.ShapeDtypeStruct((HEADS, S, TOPK * SC_LANES), jnp.float32),
        ],
    )(x, pkw[0], pkw[1], pk_keys_t[0], pk_keys_t[1], e0, e1)

    sc_gather = pl.kernel(
        _sc_gather_body,
        mesh=plsc.VectorSubcoreMesh(core_axis_name="c", subcore_axis_name="s"),
        out_type=jax.ShapeDtypeStruct((SC_BAGS, DIM), jnp.float32),
        scratch_types=[
            pltpu.VMEM((SC_BPW * TOPK,), jnp.int32),
            pltpu.VMEM((2, SC_CH * TOPK, SC_LANES), jnp.float32),
            pltpu.VMEM((2, SC_CH * TOPK, DIM), jnp.float32),
            pltpu.VMEM((SC_CH, DIM), jnp.float32),
            pltpu.SemaphoreType.DMA,
            pltpu.SemaphoreType.DMA,
            pltpu.SemaphoreType.DMA,
            pltpu.SemaphoreType.DMA,
            pltpu.SemaphoreType.DMA,
        ],
    )
    k = sc_gather(idx.reshape(SC_BAGS * TOPK),
                  wwide.reshape(SC_BAGS * TOPK, SC_LANES),
                  keys_emb)
    k = k.reshape(HEADS, S, DIM)

    v = pl.pallas_call(
        _vbuild_body,
        grid=(HEADS, S // S_BLK_G),
        in_specs=[
            pl.BlockSpec((1, S_BLK_G, TOPK), lambda h, sb: (h, sb, 0)),
            pl.BlockSpec((1, S_BLK_G, TOPK), lambda h, sb: (h, sb, 0)),
            pl.BlockSpec((NUM_KV, DIM), lambda h, sb: (h, 0)),
        ],
        out_specs=pl.BlockSpec((1, S_BLK_G, DIM), lambda h, sb: (h, sb, 0)),
        out_shape=jax.ShapeDtypeStruct((HEADS, S, DIM), jnp.bfloat16),
    )(idx, scores, values_emb)

    out = pl.pallas_call(
        _attn_body,
        grid=(HEADS, S // S_BLK_A),
        in_specs=[
            pl.BlockSpec((S_BLK_A, DIM), lambda h, qb: (qb, 0)),
            pl.BlockSpec((DIM, DIM), lambda h, qb: (0, h)),
            pl.BlockSpec((1, S, DIM), lambda h, qb: (h, 0, 0)),
            pl.BlockSpec((1, S, DIM), lambda h, qb: (h, 0, 0)),
            pl.BlockSpec((DIM, DIM), lambda h, qb: (h, 0)),
        ],
        out_specs=pl.BlockSpec((S, DIM), lambda h, qb: (0, 0)),
        out_shape=jax.ShapeDtypeStruct((S, DIM), jnp.float32),
        scratch_shapes=[pltpu.VMEM((S_BLK_A, S), jnp.float32)],
    )(x, Wq, k, v, Wout)

    return out[None]


# speedup vs baseline: 1.0422x; 1.0422x over previous
import jax
import jax.numpy as jnp
from jax import lax
from jax.experimental import pallas as pl
from jax.experimental.pallas import tpu as pltpu
from jax.experimental.pallas import tpu_sc as plsc

DIM = 768
HEADS = 12
NUM_KV = 784
NUM_KEYS = 28
TOPK = 8
DIM_KEY = 48
S = 2048
S_BLK_R = 512
S_BLK_G = 512
S_BLK_A = 256
PAD = 896

_NEG = -1e30
_HIGHEST = lax.Precision.HIGHEST


def _top8(c):
    j = lax.broadcasted_iota(jnp.int32, c.shape, 1)
    vals, idxs = [], []
    for _ in range(TOPK):
        m = jnp.max(c, axis=1, keepdims=True)
        sel = jnp.min(jnp.where(c == m, j, jnp.int32(1 << 30)), axis=1, keepdims=True)
        vals.append(m)
        idxs.append(sel)
        c = jnp.where(j == sel, _NEG, c)
    return jnp.concatenate(vals, axis=1), jnp.concatenate(idxs, axis=1)


def _router_body(x_ref, pkw0_ref, pkw1_ref, k0_ref, k1_ref, e0_ref, e1_ref,
                 scores_ref, idx_ref, wwide_ref):
    x = x_ref[...].astype(jnp.bfloat16)
    qpk0 = jnp.dot(x, pkw0_ref[0], preferred_element_type=jnp.float32)
    qpk1 = jnp.dot(x, pkw1_ref[0], preferred_element_type=jnp.float32)
    sim0 = jnp.dot(qpk0.astype(jnp.bfloat16), k0_ref[0],
                   preferred_element_type=jnp.float32)
    sim1 = jnp.dot(qpk1.astype(jnp.bfloat16), k1_ref[0],
                   preferred_element_type=jnp.float32)

    c = (jnp.dot(sim0, e0_ref[...], precision=_HIGHEST, preferred_element_type=jnp.float32)
         + jnp.dot(sim1, e1_ref[...], precision=_HIGHEST, preferred_element_type=jnp.float32))
    j2 = lax.broadcasted_iota(jnp.int32, (S_BLK_R, PAD), 1)
    c = jnp.where(j2 < NUM_KV, c, _NEG)

    jf = j2.astype(jnp.float32)
    vals, idxs = [], []
    for _ in range(TOPK):
        m = jnp.max(c, axis=1, keepdims=True)
        eq = c == m
        idx_f = jnp.min(jnp.where(eq, jf, 1e9), axis=1, keepdims=True)
        c = jnp.where(eq, _NEG, c)
        vals.append(m)
        idxs.append(idx_f)
    scores = jnp.concatenate(vals, axis=1)
    idx = jnp.concatenate(idxs, axis=1).astype(jnp.int32)

    m8 = jnp.max(scores, axis=1, keepdims=True)
    e = jnp.exp(scores - m8)
    p = e / jnp.sum(e, axis=1, keepdims=True)
    scores_ref[0] = p
    idx_ref[0] = idx + pl.program_id(0) * NUM_KV
    wwide_ref[0] = jnp.concatenate(
        [jnp.broadcast_to(p[:, t:t + 1], (S_BLK_R, SC_LANES)) for t in range(TOPK)],
        axis=1)


SC_NW = 32
SC_BAGS = HEADS * S
SC_BPW = SC_BAGS // SC_NW
SC_LANES = 16
SC_CH = 4
SC_NCH = SC_BPW // SC_CH


def _sc_gather_body(idx_hbm, w_hbm, kt_hbm, vt_hbm, kout_hbm, vout_hbm,
                    idx_v, w_v, rowsk_v, rowsv_v, outk_v, outv_v,
                    semk0, semk1, semv0, semv1, semo0, semo1, semw0, semw1):
    wid = lax.axis_index("s") * 2 + lax.axis_index("c")
    base = wid * SC_BPW
    pltpu.sync_copy(idx_hbm.at[pl.ds(base * TOPK, SC_BPW * TOPK)], idx_v)
    semks = (semk0, semk1)
    semvs = (semv0, semv1)
    semos = (semo0, semo1)
    semws = (semw0, semw1)

    def fire(ci, buf):
        sl = idx_v.at[pl.ds(ci * SC_CH * TOPK, SC_CH * TOPK)]
        ck = pltpu.async_copy(kt_hbm.at[sl], rowsk_v.at[buf], semks[buf])
        cv = pltpu.async_copy(vt_hbm.at[sl], rowsv_v.at[buf], semvs[buf])
        cw = pltpu.async_copy(
            w_hbm.at[pl.ds((base + ci * SC_CH) * TOPK, SC_CH * TOPK), :],
            w_v.at[buf], semws[buf])
        return ck, cv, cw

    def consume(ci, buf):
        ws = [w_v[buf, l, :] for l in range(SC_CH * TOPK)]

        def dchunk(c, _):
            for u in range(2):
                cc = c * 2 + u
                for b in range(SC_CH):
                    acck = ws[b * TOPK] * rowsk_v[buf, b * TOPK, pl.ds(cc * SC_LANES, SC_LANES)]
                    accv = ws[b * TOPK] * rowsv_v[buf, b * TOPK, pl.ds(cc * SC_LANES, SC_LANES)]
                    for t in range(1, TOPK):
                        w = ws[b * TOPK + t]
                        acck = acck + w * rowsk_v[buf, b * TOPK + t, pl.ds(cc * SC_LANES, SC_LANES)]
                        accv = accv + w * rowsv_v[buf, b * TOPK + t, pl.ds(cc * SC_LANES, SC_LANES)]
                    outk_v[buf, b, pl.ds(cc * SC_LANES, SC_LANES)] = acck
                    outv_v[buf, b, pl.ds(cc * SC_LANES, SC_LANES)] = accv
            return 0

        lax.fori_loop(0, DIM // SC_LANES // 2, dchunk, 0)
        co1 = pltpu.async_copy(outk_v.at[buf], kout_hbm.at[pl.ds(base + ci * SC_CH, SC_CH)],
                               semos[buf])
        co2 = pltpu.async_copy(outv_v.at[buf], vout_hbm.at[pl.ds(base + ci * SC_CH, SC_CH)],
                               semos[buf])
        return co1, co2

    fire(0, 0)

    def pair(i, _):
        ci0 = i * 2
        fire(ci0 + 1, 1)
        pltpu.make_async_copy(kt_hbm.at[idx_v.at[pl.ds(0, SC_CH * TOPK)]],
                              rowsk_v.at[0], semk0).wait()
        pltpu.make_async_copy(vt_hbm.at[idx_v.at[pl.ds(0, SC_CH * TOPK)]],
                              rowsv_v.at[0], semv0).wait()
        pltpu.make_async_copy(w_hbm.at[pl.ds(0, SC_CH * TOPK), :],
                              w_v.at[0], semw0).wait()

        @pl.when(i > 0)
        def _():
            pltpu.make_async_copy(outk_v.at[0], kout_hbm.at[pl.ds(base, SC_CH)], semo0).wait()
            pltpu.make_async_copy(outv_v.at[0], vout_hbm.at[pl.ds(base, SC_CH)], semo0).wait()

        consume(ci0, 0)

        @pl.when(i < SC_NCH // 2 - 1)
        def _():
            fire(ci0 + 2, 0)

        pltpu.make_async_copy(kt_hbm.at[idx_v.at[pl.ds(0, SC_CH * TOPK)]],
                              rowsk_v.at[1], semk1).wait()
        pltpu.make_async_copy(vt_hbm.at[idx_v.at[pl.ds(0, SC_CH * TOPK)]],
                              rowsv_v.at[1], semv1).wait()
        pltpu.make_async_copy(w_hbm.at[pl.ds(0, SC_CH * TOPK), :],
                              w_v.at[1], semw1).wait()

        @pl.when(i > 0)
        def _():
            pltpu.make_async_copy(outk_v.at[1], kout_hbm.at[pl.ds(base, SC_CH)], semo1).wait()
            pltpu.make_async_copy(outv_v.at[1], vout_hbm.at[pl.ds(base, SC_CH)], semo1).wait()

        consume(ci0 + 1, 1)
        return 0

    lax.fori_loop(0, SC_NCH // 2, pair, 0)
    pltpu.make_async_copy(outk_v.at[0], kout_hbm.at[pl.ds(base, SC_CH)], semo0).wait()
    pltpu.make_async_copy(outv_v.at[0], vout_hbm.at[pl.ds(base, SC_CH)], semo0).wait()
    pltpu.make_async_copy(outk_v.at[1], kout_hbm.at[pl.ds(base, SC_CH)], semo1).wait()
    pltpu.make_async_copy(outv_v.at[1], vout_hbm.at[pl.ds(base, SC_CH)], semo1).wait()


def _attn_body(x_ref, wq_ref, k_ref, v_ref, wout_ref, out_ref, sim_ref):
    h = pl.program_id(0)
    qb = pl.program_id(1)
    nkb = qb + 1
    q = jnp.dot(x_ref[...].astype(jnp.bfloat16), wq_ref[...].astype(jnp.bfloat16),
                preferred_element_type=jnp.float32)
    q = (q * (DIM ** -0.5)).astype(jnp.bfloat16)

    def qk_step(kb, _):
        kblk = k_ref[0, pl.ds(kb * S_BLK_A, S_BLK_A), :].astype(jnp.bfloat16)
        sim_ref[:, pl.ds(kb * S_BLK_A, S_BLK_A)] = lax.dot_general(
            q, kblk, (((1,), (1,)), ((), ())), preferred_element_type=jnp.float32)
        return 0

    lax.fori_loop(0, nkb, qk_step, 0)

    col = lax.broadcasted_iota(jnp.int32, (S_BLK_A, S), 1)
    row = lax.broadcasted_iota(jnp.int32, (S_BLK_A, S), 0) + qb * S_BLK_A
    s = jnp.where(col > row, _NEG, sim_ref[...])
    m = jnp.max(s, axis=1, keepdims=True)
    p = jnp.exp(s - m)
    p = p / jnp.sum(p, axis=1, keepdims=True)
    sim_ref[...] = p

    def av_step(kb, o):
        pblk = sim_ref[:, pl.ds(kb * S_BLK_A, S_BLK_A)].astype(jnp.bfloat16)
        vblk = v_ref[0, pl.ds(kb * S_BLK_A, S_BLK_A), :].astype(jnp.bfloat16)
        return o + jnp.dot(pblk, vblk, preferred_element_type=jnp.float32)

    o = lax.fori_loop(0, nkb, av_step, jnp.zeros((S_BLK_A, DIM), jnp.float32))
    proj = jnp.dot(o.astype(jnp.bfloat16), wout_ref[...].astype(jnp.bfloat16),
                   preferred_element_type=jnp.float32)

    @pl.when(h == 0)
    def _():
        out_ref[pl.ds(qb * S_BLK_A, S_BLK_A), :] = proj

    @pl.when(h > 0)
    def _():
        out_ref[pl.ds(qb * S_BLK_A, S_BLK_A), :] += proj


@jax.jit
def kernel(inputs, Wq, keys_emb, values_emb, pk_Wq, pk_keys, Wout):
    x = inputs[0]
    pk_keys_t = jnp.transpose(pk_keys, (0, 2, 3, 1)).astype(jnp.bfloat16)
    pkw = jnp.transpose(pk_Wq.reshape(DIM, 2, HEADS, DIM_KEY),
                        (1, 2, 0, 3)).astype(jnp.bfloat16)

    r = lax.broadcasted_iota(jnp.int32, (NUM_KEYS, PAD), 0)
    j = lax.broadcasted_iota(jnp.int32, (NUM_KEYS, PAD), 1)
    valid = j < NUM_KV
    e0 = ((j % NUM_KEYS == r) & valid).astype(jnp.float32)
    e1 = ((j // NUM_KEYS == r) & valid).astype(jnp.float32)

    scores, idx, wwide = pl.pallas_call(
        _router_body,
        grid=(HEADS, S // S_BLK_R),
        in_specs=[
            pl.BlockSpec((S_BLK_R, DIM), lambda h, sb: (sb, 0)),
            pl.BlockSpec((1, DIM, DIM_KEY), lambda h, sb: (h, 0, 0)),
            pl.BlockSpec((1, DIM, DIM_KEY), lambda h, sb: (h, 0, 0)),
            pl.BlockSpec((1, DIM_KEY, NUM_KEYS), lambda h, sb: (h, 0, 0)),
            pl.BlockSpec((1, DIM_KEY, NUM_KEYS), lambda h, sb: (h, 0, 0)),
            pl.BlockSpec((NUM_KEYS, PAD), lambda h, sb: (0, 0)),
            pl.BlockSpec((NUM_KEYS, PAD), lambda h, sb: (0, 0)),
        ],
        out_specs=[
            pl.BlockSpec((1, S_BLK_R, TOPK), lambda h, sb: (h, sb, 0)),
            pl.BlockSpec((1, S_BLK_R, TOPK), lambda h, sb: (h, sb, 0)),
            pl.BlockSpec((1, S_BLK_R, TOPK * SC_LANES), lambda h, sb: (h, sb, 0)),
        ],
        out_shape=[
            jax.ShapeDtypeStruct((HEADS, S, TOPK), jnp.float32),
            jax.ShapeDtypeStruct((HEADS, S, TOPK), jnp.int32),
            jax.ShapeDtypeStruct((HEADS, S, TOPK * SC_LANES), jnp.float32),
        ],
    )(x, pkw[0], pkw[1], pk_keys_t[0], pk_keys_t[1], e0, e1)

    sc_gather = pl.kernel(
        _sc_gather_body,
        mesh=plsc.VectorSubcoreMesh(core_axis_name="c", subcore_axis_name="s"),
        out_type=[
            jax.ShapeDtypeStruct((SC_BAGS, DIM), jnp.float32),
            jax.ShapeDtypeStruct((SC_BAGS, DIM), jnp.float32),
        ],
        scratch_types=[
            pltpu.VMEM((SC_BPW * TOPK,), jnp.int32),
            pltpu.VMEM((2, SC_CH * TOPK, SC_LANES), jnp.float32),
            pltpu.VMEM((2, SC_CH * TOPK, DIM), jnp.float32),
            pltpu.VMEM((2, SC_CH * TOPK, DIM), jnp.float32),
            pltpu.VMEM((2, SC_CH, DIM), jnp.float32),
            pltpu.VMEM((2, SC_CH, DIM), jnp.float32),
            pltpu.SemaphoreType.DMA,
            pltpu.SemaphoreType.DMA,
            pltpu.SemaphoreType.DMA,
            pltpu.SemaphoreType.DMA,
            pltpu.SemaphoreType.DMA,
            pltpu.SemaphoreType.DMA,
            pltpu.SemaphoreType.DMA,
            pltpu.SemaphoreType.DMA,
        ],
    )
    k, v = sc_gather(idx.reshape(SC_BAGS * TOPK),
                     wwide.reshape(SC_BAGS * TOPK, SC_LANES),
                     keys_emb, values_emb)
    k = k.reshape(HEADS, S, DIM)
    v = v.reshape(HEADS, S, DIM)

    out = pl.pallas_call(
        _attn_body,
        grid=(HEADS, S // S_BLK_A),
        in_specs=[
            pl.BlockSpec((S_BLK_A, DIM), lambda h, qb: (qb, 0)),
            pl.BlockSpec((DIM, DIM), lambda h, qb: (0, h)),
            pl.BlockSpec((1, S, DIM), lambda h, qb: (h, 0, 0)),
            pl.BlockSpec((1, S, DIM), lambda h, qb: (h, 0, 0)),
            pl.BlockSpec((DIM, DIM), lambda h, qb: (h, 0)),
        ],
        out_specs=pl.BlockSpec((S, DIM), lambda h, qb: (0, 0)),
        out_shape=jax.ShapeDtypeStruct((S, DIM), jnp.float32),
        scratch_shapes=[pltpu.VMEM((S_BLK_A, S), jnp.float32)],
    )(x, Wq, k, v, Wout)

    return out[None]
